# Initial kernel scaffold; baseline (speedup 1.0000x reference)
import jax
import jax.numpy as jnp
from jax.experimental import pallas as pl


def kernel(inputs, target):
    raise NotImplementedError("write your pallas kernel here")


# speedup vs baseline: 1.0069x; 1.0069x over previous
import functools

import jax
import jax.numpy as jnp
from jax.experimental import pallas as pl
from jax.experimental.pallas import tpu as pltpu

B = 32
N = 512 * 512
SAMPLES_PER_BLOCK = 4
GRID = B // SAMPLES_PER_BLOCK


def _body(x_ref, t_ref, out_ref, acc_ref):
    i = pl.program_id(0)

    @pl.when(i == 0)
    def _init():
        acc_ref[0] = 0.0
        acc_ref[1] = 0.0

    x = x_ref[...]
    t = t_ref[...].astype(jnp.float32)

    ax = jnp.abs(x)
    e = jnp.exp(-ax)
    inv = 1.0 / (1.0 + e)
    s = jnp.where(x >= 0.0, inv, e * inv)

    s_sum = jnp.sum(s, axis=(1, 2))
    st_sum = jnp.sum(s * t, axis=(1, 2))
    t_sum = jnp.sum(t, axis=(1, 2))
    dice = 1.0 - (2.0 * st_sum + 1.0) / (s_sum + t_sum + 1.0)

    bce = jnp.maximum(x, 0.0) - x * t + jnp.log1p(e)

    acc_ref[0] += jnp.sum(dice)
    acc_ref[1] += jnp.sum(bce)

    @pl.when(i == GRID - 1)
    def _fin():
        out_ref[0] = acc_ref[0] / B + acc_ref[1] / (B * N)


@jax.jit
def kernel(inputs, target):
    x = inputs.reshape(B, 512, 512)
    return pl.pallas_call(
        _body,
        grid=(GRID,),
        in_specs=[
            pl.BlockSpec((SAMPLES_PER_BLOCK, 512, 512), lambda i: (i, 0, 0)),
            pl.BlockSpec((SAMPLES_PER_BLOCK, 512, 512), lambda i: (i, 0, 0)),
        ],
        out_specs=pl.BlockSpec(memory_space=pltpu.SMEM),
        out_shape=jax.ShapeDtypeStruct((1,), jnp.float32),
        scratch_shapes=[pltpu.SMEM((2,), jnp.float32)],
        compiler_params=pltpu.CompilerParams(
            dimension_semantics=("arbitrary",),
        ),
    )(x, target)
